# Initial kernel scaffold; baseline (speedup 1.0000x reference)
import jax
import jax.numpy as jnp
from jax.experimental import pallas as pl


def kernel(edge_index, edge_weight, interaction_preference, interaction_embedding):
    raise NotImplementedError("write your pallas kernel here")


# speedup vs baseline: 1.6741x; 1.6741x over previous
import functools

import jax
import jax.numpy as jnp
from jax import lax
from jax.experimental import pallas as pl
from jax.experimental.pallas import tpu as pltpu
from jax.experimental.pallas import tpu_sc as plsc

N_USERS_K = 5000
N_ITEMS_K = 5000
N_NODES_K = N_USERS_K + N_ITEMS_K
E_K = 320000
D_K = 128
ALPHA_K = 0.5

NC, NS, L = 2, 16, 16
NW = NC * NS
NPT = 320
N_PAD = NW * NPT
CHUNK = 128
JG = D_K // L


def _build_ego_body(pref_ref, emb_ref, out_ref):
    out_ref[0:N_USERS_K, :] = pref_ref[...]
    e = emb_ref[...]
    n = jnp.sqrt(jnp.sum(e * e, axis=1, keepdims=True))
    out_ref[N_USERS_K:N_NODES_K, :] = e / jnp.maximum(n, 1e-12)
    out_ref[N_NODES_K:N_PAD, :] = jnp.zeros((N_PAD - N_NODES_K, D_K), jnp.float32)


def _build_ego(pref, emb):
    return pl.pallas_call(
        _build_ego_body,
        out_shape=jax.ShapeDtypeStruct((N_PAD, D_K), jnp.float32),
    )(pref, emb)


_mesh = plsc.VectorSubcoreMesh(core_axis_name="c", subcore_axis_name="s")


@functools.partial(
    pl.kernel,
    out_type=jax.ShapeDtypeStruct((N_PAD, D_K), jnp.float32),
    mesh=_mesh,
    scratch_types=[
        pltpu.VMEM((NPT, D_K), jnp.float32),
        pltpu.VMEM((CHUNK,), jnp.int32),
        pltpu.VMEM((CHUNK + L,), jnp.float32),
        pltpu.VMEM((CHUNK + L,), jnp.int32),
        pltpu.VMEM((CHUNK, D_K), jnp.float32),
        pltpu.VMEM((48,), jnp.int32),
        pltpu.SemaphoreType.DMA,
    ],
)
def _sc_layer(ego, srcs, ws, dsts, starts, out, acc, sidx, wbuf, dbuf, gbuf,
              stv, sem):
    cid = lax.axis_index("c")
    sid = lax.axis_index("s")
    tid = sid * NC + cid
    base = tid * NPT

    pltpu.sync_copy(starts, stv)
    start = stv[pl.ds(tid, L)][0]
    end = stv[pl.ds(tid + 1, L)][0]
    c_lo = start // CHUNK
    c_hi = (end + CHUNK - 1) // CHUNK

    pltpu.sync_copy(ego.at[pl.ds(base, NPT)], acc)

    def _scale_row(r, carry):
        for j in range(JG):
            sl = pl.ds(j * L, L)
            acc[r, sl] = acc[r, sl] * ALPHA_K
        return carry

    lax.fori_loop(0, NPT, _scale_row, 0)

    def _chunk(cix, carry):
        off = cix * CHUNK
        pltpu.sync_copy(srcs.at[pl.ds(off, CHUNK)], sidx)
        pltpu.sync_copy(ws.at[pl.ds(off, CHUNK)], wbuf.at[pl.ds(0, CHUNK)])
        pltpu.sync_copy(dsts.at[pl.ds(off, CHUNK)], dbuf.at[pl.ds(0, CHUNK)])
        pltpu.async_copy(ego.at[sidx], gbuf, sem).wait()

        def _edge(e, ecarry):
            d = dbuf[pl.ds(e, L)][0]
            ok = jnp.logical_and(d >= base, d < base + NPT)

            @pl.when(ok)
            def _():
                w = wbuf[pl.ds(e, L)][0]
                wv = jnp.full((L,), w, jnp.float32)
                dl = d - base
                for j in range(JG):
                    sl = pl.ds(j * L, L)
                    plsc.addupdate(acc.at[dl, sl], gbuf[e, sl] * wv)

            return ecarry

        lax.fori_loop(0, CHUNK, _edge, 0)
        return carry

    lax.fori_loop(c_lo, c_hi, _chunk, 0)

    pltpu.sync_copy(acc, out.at[pl.ds(base, NPT)])


def kernel(edge_index, edge_weight, interaction_preference, interaction_embedding):
    src = edge_index[0].astype(jnp.int32)
    dst = edge_index[1].astype(jnp.int32)
    w = edge_weight[:, 0].astype(jnp.float32)

    dst_s, src_s, w_s = lax.sort((dst, src, w), num_keys=1)
    bounds = jnp.arange(33, dtype=jnp.int32) * NPT
    starts = jnp.searchsorted(dst_s, bounds).astype(jnp.int32)
    starts48 = jnp.concatenate([starts, jnp.full((15,), E_K, jnp.int32)])

    ego = _build_ego(interaction_preference, interaction_embedding)
    ego = _sc_layer(ego, src_s, w_s, dst_s, starts48)
    ego = _sc_layer(ego, src_s, w_s, dst_s, starts48)
    return ego[:N_NODES_K]


# speedup vs baseline: 2.0778x; 1.2411x over previous
import functools

import jax
import jax.numpy as jnp
from jax import lax
from jax.experimental import pallas as pl
from jax.experimental.pallas import tpu as pltpu
from jax.experimental.pallas import tpu_sc as plsc

N_USERS_K = 5000
N_ITEMS_K = 5000
N_NODES_K = N_USERS_K + N_ITEMS_K
E_K = 320000
D_K = 128
ALPHA_K = 0.5

NC, NS, L = 2, 16, 16
NW = NC * NS
NPT = 320
N_PAD = NW * NPT
CHUNK = 128
NCHUNK = E_K // CHUNK
JG = D_K // L
EROW = 2 * CHUNK
SBC = 64


def _build_ego_body(pref_ref, emb_ref, out_ref):
    out_ref[0:N_USERS_K, :] = pref_ref[...]
    e = emb_ref[...]
    n = jnp.sqrt(jnp.sum(e * e, axis=1, keepdims=True))
    out_ref[N_USERS_K:N_NODES_K, :] = e / jnp.maximum(n, 1e-12)
    out_ref[N_NODES_K:N_PAD, :] = jnp.zeros((N_PAD - N_NODES_K, D_K), jnp.float32)


def _build_ego(pref, emb):
    return pl.pallas_call(
        _build_ego_body,
        out_shape=jax.ShapeDtypeStruct((N_PAD, D_K), jnp.float32),
    )(pref, emb)


_mesh = plsc.VectorSubcoreMesh(core_axis_name="c", subcore_axis_name="s")


@functools.partial(
    pl.kernel,
    out_type=jax.ShapeDtypeStruct((N_PAD, D_K), jnp.float32),
    mesh=_mesh,
    scratch_types=[
        pltpu.VMEM((NPT, D_K), jnp.float32),
        pltpu.VMEM((SBC * EROW,), jnp.int32),
        pltpu.VMEM((SBC * CHUNK,), jnp.float32),
        pltpu.VMEM((CHUNK, D_K), jnp.float32),
        pltpu.VMEM((CHUNK, D_K), jnp.float32),
        pltpu.VMEM((48,), jnp.int32),
        pltpu.SemaphoreType.DMA,
        pltpu.SemaphoreType.DMA,
        pltpu.SemaphoreType.DMA,
    ],
)
def _sc_layer(ego, edata, wdata, starts, out, acc, ebuf, wsbuf, gbuf0, gbuf1,
              stv, sem_e, sem_g0, sem_g1):
    cid = lax.axis_index("c")
    sid = lax.axis_index("s")
    tid = sid * NC + cid
    base = tid * NPT

    pltpu.sync_copy(starts, stv)
    start = stv[pl.ds(tid, L)][0]
    end = stv[pl.ds(tid + 1, L)][0]
    c_lo = start // CHUNK
    c_hi = (end + CHUNK - 1) // CHUNK
    nchunks = c_hi - c_lo

    pltpu.sync_copy(ego.at[pl.ds(base, NPT)], acc)

    def _scale_row(r, carry):
        for j in range(JG):
            sl = pl.ds(j * L, L)
            acc[r, sl] = acc[r, sl] * ALPHA_K
        return carry

    lax.fori_loop(0, NPT, _scale_row, 0)

    def _start_gather(k, gbuf, sem):
        idx = ebuf.at[pl.ds(k * EROW + CHUNK, CHUNK)]
        return pltpu.async_copy(ego.at[idx], gbuf, sem)

    def _compute(k, gbuf):
        def _edge(e, ecarry):
            d = ebuf[pl.ds(k * EROW + e, L)][0]
            ok = jnp.logical_and(d >= base, d < base + NPT)

            @pl.when(ok)
            def _():
                w = wsbuf[pl.ds(k * CHUNK + e, L)][0]
                wv = jnp.full((L,), w, jnp.float32)
                dl = d - base
                for j in range(JG):
                    sl = pl.ds(j * L, L)
                    plsc.addupdate(acc.at[dl, sl], gbuf[e, sl] * wv)

            return ecarry

        lax.fori_loop(0, CHUNK, _edge, 0)

    cs0 = (c_lo // SBC) * SBC

    def _super_block(s, carry):
        cs = cs0 + s * SBC
        k_begin = jnp.maximum(c_lo - cs, 0)
        k_end = jnp.minimum(c_hi - cs, SBC)
        pltpu.async_copy(edata.at[pl.ds(cs * EROW, SBC * EROW)], ebuf, sem_e).wait()
        pltpu.async_copy(wdata.at[pl.ds(cs * CHUNK, SBC * CHUNK)], wsbuf, sem_e).wait()

        @pl.when(k_begin < k_end)
        def _():
            _start_gather(k_begin, gbuf0, sem_g0)

        def _pair(q, pcarry):
            k0 = k_begin + 2 * q
            k1 = k0 + 1

            @pl.when(k1 < k_end)
            def _():
                _start_gather(k1, gbuf1, sem_g1)

            pltpu.make_async_copy(ego.at[ebuf.at[pl.ds(0, CHUNK)]],
                                  gbuf0, sem_g0).wait()
            _compute(k0, gbuf0)

            @pl.when(k0 + 2 < k_end)
            def _():
                _start_gather(k0 + 2, gbuf0, sem_g0)

            @pl.when(k1 < k_end)
            def _():
                pltpu.make_async_copy(
                    ego.at[ebuf.at[pl.ds(0, CHUNK)]],
                    gbuf1, sem_g1).wait()
                _compute(k1, gbuf1)

            return pcarry

        lax.fori_loop(0, (k_end - k_begin + 1) // 2, _pair, 0)
        return carry

    nsb = (c_hi - cs0 + SBC - 1) // SBC
    lax.fori_loop(0, nsb, _super_block, 0)

    pltpu.sync_copy(acc, out.at[pl.ds(base, NPT)])


def kernel(edge_index, edge_weight, interaction_preference, interaction_embedding):
    src = edge_index[0].astype(jnp.int32)
    dst = edge_index[1].astype(jnp.int32)
    w = edge_weight[:, 0].astype(jnp.float32)

    dst_s, src_s, w_s = lax.sort((dst, src, w), num_keys=1)
    bounds = jnp.arange(33, dtype=jnp.int32) * NPT
    starts = jnp.searchsorted(dst_s, bounds).astype(jnp.int32)
    starts48 = jnp.concatenate([starts, jnp.full((15,), E_K, jnp.int32)])

    edata = jnp.concatenate(
        [dst_s.reshape(NCHUNK, CHUNK),
         src_s.reshape(NCHUNK, CHUNK)], axis=1)
    edata = jnp.concatenate(
        [edata, jnp.zeros((SBC, EROW), jnp.int32)], axis=0).reshape(-1)
    wdata = jnp.concatenate([w_s, jnp.zeros((SBC * CHUNK,), jnp.float32)])

    ego = _build_ego(interaction_preference, interaction_embedding)
    ego = _sc_layer(ego, edata, wdata, starts48)
    ego = _sc_layer(ego, edata, wdata, starts48)
    return ego[:N_NODES_K]


# speedup vs baseline: 3.2180x; 1.5488x over previous
import functools

import jax
import jax.numpy as jnp
from jax import lax
from jax.experimental import pallas as pl
from jax.experimental.pallas import tpu as pltpu
from jax.experimental.pallas import tpu_sc as plsc

N_USERS_K = 5000
N_ITEMS_K = 5000
N_NODES_K = N_USERS_K + N_ITEMS_K
E_K = 320000
D_K = 128
ALPHA_K = 0.5

NC, NS, L = 2, 16, 16
NW = NC * NS
NPT = 320
N_PAD = NW * NPT
CHUNK = 128
NCHUNK = E_K // CHUNK
JG = D_K // L
EROW = 2 * CHUNK
SBC = 64


def _build_ego_body(pref_ref, emb_ref, out_ref):
    out_ref[0:N_USERS_K, :] = pref_ref[...]
    e = emb_ref[...]
    n = jnp.sqrt(jnp.sum(e * e, axis=1, keepdims=True))
    out_ref[N_USERS_K:N_NODES_K, :] = e / jnp.maximum(n, 1e-12)
    out_ref[N_NODES_K:N_PAD, :] = jnp.zeros((N_PAD - N_NODES_K, D_K), jnp.float32)


def _build_ego(pref, emb):
    return pl.pallas_call(
        _build_ego_body,
        out_shape=jax.ShapeDtypeStruct((N_PAD, D_K), jnp.float32),
    )(pref, emb)


_mesh = plsc.VectorSubcoreMesh(core_axis_name="c", subcore_axis_name="s")


@functools.partial(
    pl.kernel,
    out_type=jax.ShapeDtypeStruct((N_PAD, D_K), jnp.float32),
    mesh=_mesh,
    scratch_types=[
        pltpu.VMEM((NPT, D_K), jnp.float32),
        pltpu.VMEM((SBC * EROW,), jnp.int32),
        pltpu.VMEM((SBC * CHUNK,), jnp.float32),
        pltpu.VMEM((CHUNK, D_K), jnp.float32),
        pltpu.VMEM((CHUNK, D_K), jnp.float32),
        pltpu.VMEM((48,), jnp.int32),
        pltpu.SemaphoreType.DMA,
        pltpu.SemaphoreType.DMA,
        pltpu.SemaphoreType.DMA,
    ],
)
def _sc_layer(ego, edata, wdata, starts, out, acc, ebuf, wsbuf, gbuf0, gbuf1,
              stv, sem_e, sem_g0, sem_g1):
    cid = lax.axis_index("c")
    sid = lax.axis_index("s")
    tid = sid * NC + cid
    base = tid * NPT

    pltpu.sync_copy(starts, stv)
    start = stv[pl.ds(tid, L)][0]
    end = stv[pl.ds(tid + 1, L)][0]
    c_lo = start // CHUNK
    c_hi = (end + CHUNK - 1) // CHUNK
    nchunks = c_hi - c_lo

    pltpu.sync_copy(ego.at[pl.ds(base, NPT)], acc)

    def _scale_row(r, carry):
        for j in range(JG):
            sl = pl.ds(j * L, L)
            acc[r, sl] = acc[r, sl] * ALPHA_K
        return carry

    lax.fori_loop(0, NPT, _scale_row, 0)

    def _start_gather(k, gbuf, sem):
        idx = ebuf.at[pl.ds(k * EROW + CHUNK, CHUNK)]
        return pltpu.async_copy(ego.at[idx], gbuf, sem)

    def _compute(k, gbuf):
        def _edge(e, d_carry):
            d = d_carry
            ok = jnp.logical_and(d >= base, d < base + NPT)

            @pl.when(ok)
            def _():
                w = wsbuf[pl.ds(k * CHUNK + e, L)][0]
                wv = jnp.full((L,), w, jnp.float32)
                dl = d - base
                vals = [gbuf[e, pl.ds(j * L, L)] * wv for j in range(JG)]
                for j in range(JG):
                    plsc.addupdate(acc.at[dl, pl.ds(j * L, L)], vals[j])

            return ebuf[pl.ds(k * EROW + e + 1, L)][0]

        d0 = ebuf[pl.ds(k * EROW, L)][0]
        lax.fori_loop(0, CHUNK, _edge, d0)

    cs0 = (c_lo // SBC) * SBC

    def _super_block(s, carry):
        cs = cs0 + s * SBC
        k_begin = jnp.maximum(c_lo - cs, 0)
        k_end = jnp.minimum(c_hi - cs, SBC)
        pltpu.async_copy(edata.at[pl.ds(cs * EROW, SBC * EROW)], ebuf, sem_e).wait()
        pltpu.async_copy(wdata.at[pl.ds(cs * CHUNK, SBC * CHUNK)], wsbuf, sem_e).wait()

        @pl.when(k_begin < k_end)
        def _():
            _start_gather(k_begin, gbuf0, sem_g0)

        def _pair(q, pcarry):
            k0 = k_begin + 2 * q
            k1 = k0 + 1

            @pl.when(k1 < k_end)
            def _():
                _start_gather(k1, gbuf1, sem_g1)

            pltpu.make_async_copy(ego.at[ebuf.at[pl.ds(0, CHUNK)]],
                                  gbuf0, sem_g0).wait()
            _compute(k0, gbuf0)

            @pl.when(k0 + 2 < k_end)
            def _():
                _start_gather(k0 + 2, gbuf0, sem_g0)

            @pl.when(k1 < k_end)
            def _():
                pltpu.make_async_copy(
                    ego.at[ebuf.at[pl.ds(0, CHUNK)]],
                    gbuf1, sem_g1).wait()
                _compute(k1, gbuf1)

            return pcarry

        lax.fori_loop(0, (k_end - k_begin + 1) // 2, _pair, 0)
        return carry

    nsb = (c_hi - cs0 + SBC - 1) // SBC
    lax.fori_loop(0, nsb, _super_block, 0)

    pltpu.sync_copy(acc, out.at[pl.ds(base, NPT)])


def kernel(edge_index, edge_weight, interaction_preference, interaction_embedding):
    src = edge_index[0].astype(jnp.int32)
    dst = edge_index[1].astype(jnp.int32)
    w = edge_weight[:, 0].astype(jnp.float32)

    dst_s, src_s, w_s = lax.sort((dst, src, w), num_keys=1)
    bounds = jnp.arange(33, dtype=jnp.int32) * NPT
    starts = jnp.searchsorted(dst_s, bounds).astype(jnp.int32)
    starts48 = jnp.concatenate([starts, jnp.full((15,), E_K, jnp.int32)])

    edata = jnp.concatenate(
        [dst_s.reshape(NCHUNK, CHUNK),
         src_s.reshape(NCHUNK, CHUNK)], axis=1)
    edata = jnp.concatenate(
        [edata, jnp.zeros((SBC, EROW), jnp.int32)], axis=0).reshape(-1)
    wdata = jnp.concatenate([w_s, jnp.zeros((SBC * CHUNK,), jnp.float32)])

    ego = _build_ego(interaction_preference, interaction_embedding)
    ego = _sc_layer(ego, edata, wdata, starts48)
    ego = _sc_layer(ego, edata, wdata, starts48)
    return ego[:N_NODES_K]


# speedup vs baseline: 3.3296x; 1.0347x over previous
import functools

import jax
import jax.numpy as jnp
from jax import lax
from jax.experimental import pallas as pl
from jax.experimental.pallas import tpu as pltpu
from jax.experimental.pallas import tpu_sc as plsc

N_USERS_K = 5000
N_ITEMS_K = 5000
N_NODES_K = N_USERS_K + N_ITEMS_K
E_K = 320000
D_K = 128
ALPHA_K = 0.5

NC, NS, L = 2, 16, 16
NW = NC * NS
NPT = 320
N_PAD = NW * NPT
CHUNK = 128
NCHUNK = E_K // CHUNK
JG = D_K // L
EROW = 2 * CHUNK
SBC = 64


def _build_ego_body(pref_ref, emb_ref, out_ref):
    out_ref[0:N_USERS_K, :] = pref_ref[...]
    e = emb_ref[...]
    n = jnp.sqrt(jnp.sum(e * e, axis=1, keepdims=True))
    out_ref[N_USERS_K:N_NODES_K, :] = e / jnp.maximum(n, 1e-12)
    out_ref[N_NODES_K:N_PAD, :] = jnp.zeros((N_PAD - N_NODES_K, D_K), jnp.float32)


def _build_ego(pref, emb):
    return pl.pallas_call(
        _build_ego_body,
        out_shape=jax.ShapeDtypeStruct((N_PAD, D_K), jnp.float32),
    )(pref, emb)


_mesh = plsc.VectorSubcoreMesh(core_axis_name="c", subcore_axis_name="s")


@functools.partial(
    pl.kernel,
    out_type=jax.ShapeDtypeStruct((N_PAD, D_K), jnp.float32),
    mesh=_mesh,
    scratch_types=[
        pltpu.VMEM((NPT, D_K), jnp.float32),
        pltpu.VMEM((SBC * EROW,), jnp.int32),
        pltpu.VMEM((SBC * CHUNK,), jnp.float32),
        pltpu.VMEM((CHUNK, D_K), jnp.float32),
        pltpu.VMEM((CHUNK, D_K), jnp.float32),
        pltpu.VMEM((48,), jnp.int32),
        pltpu.SemaphoreType.DMA,
        pltpu.SemaphoreType.DMA,
        pltpu.SemaphoreType.DMA,
    ],
)
def _sc_layer(ego, edata, wdata, starts, out, acc, ebuf, wsbuf, gbuf0, gbuf1,
              stv, sem_e, sem_g0, sem_g1):
    cid = lax.axis_index("c")
    sid = lax.axis_index("s")
    tid = sid * NC + cid
    base = tid * NPT

    pltpu.sync_copy(starts, stv)
    start = stv[pl.ds(tid, L)][0]
    end = stv[pl.ds(tid + 1, L)][0]
    c_lo = start // CHUNK
    c_hi = (end + CHUNK - 1) // CHUNK
    nchunks = c_hi - c_lo

    pltpu.sync_copy(ego.at[pl.ds(base, NPT)], acc)

    def _scale_row(r, carry):
        for j in range(JG):
            sl = pl.ds(j * L, L)
            acc[r, sl] = acc[r, sl] * ALPHA_K
        return carry

    lax.fori_loop(0, NPT, _scale_row, 0)

    def _start_gather(k, gbuf, sem):
        idx = ebuf.at[pl.ds(k * EROW + CHUNK, CHUNK)]
        return pltpu.async_copy(ego.at[idx], gbuf, sem)

    def _compute(k, gbuf):
        def _edge(e, d_carry):
            d = d_carry
            ok = jnp.logical_and(d >= base, d < base + NPT)

            @pl.when(ok)
            def _():
                w = wsbuf[pl.ds(k * CHUNK + e, L)][0]
                wv = jnp.full((L,), w, jnp.float32)
                dl = d - base
                vals = [gbuf[e, pl.ds(j * L, L)] * wv for j in range(JG)]
                for j in range(JG):
                    plsc.addupdate(acc.at[dl, pl.ds(j * L, L)], vals[j])

            return ebuf[pl.ds(k * EROW + e + 1, L)][0]

        d0 = ebuf[pl.ds(k * EROW, L)][0]
        lax.fori_loop(0, CHUNK, _edge, d0)

    cs0 = (c_lo // SBC) * SBC

    def _super_block(s, carry):
        cs = cs0 + s * SBC
        k_begin = jnp.maximum(c_lo - cs, 0)
        k_end = jnp.minimum(c_hi - cs, SBC)
        pltpu.async_copy(edata.at[pl.ds(cs * EROW, SBC * EROW)], ebuf, sem_e).wait()
        pltpu.async_copy(wdata.at[pl.ds(cs * CHUNK, SBC * CHUNK)], wsbuf, sem_e).wait()

        @pl.when(k_begin < k_end)
        def _():
            _start_gather(k_begin, gbuf0, sem_g0)

        def _pair(q, pcarry):
            k0 = k_begin + 2 * q
            k1 = k0 + 1

            @pl.when(k1 < k_end)
            def _():
                _start_gather(k1, gbuf1, sem_g1)

            pltpu.make_async_copy(ego.at[ebuf.at[pl.ds(0, CHUNK)]],
                                  gbuf0, sem_g0).wait()
            _compute(k0, gbuf0)

            @pl.when(k0 + 2 < k_end)
            def _():
                _start_gather(k0 + 2, gbuf0, sem_g0)

            @pl.when(k1 < k_end)
            def _():
                pltpu.make_async_copy(
                    ego.at[ebuf.at[pl.ds(0, CHUNK)]],
                    gbuf1, sem_g1).wait()
                _compute(k1, gbuf1)

            return pcarry

        lax.fori_loop(0, (k_end - k_begin + 1) // 2, _pair, 0)
        return carry

    nsb = (c_hi - cs0 + SBC - 1) // SBC
    lax.fori_loop(0, nsb, _super_block, 0)

    pltpu.sync_copy(acc, out.at[pl.ds(base, NPT)])


def kernel(edge_index, edge_weight, interaction_preference, interaction_embedding):
    src = edge_index[0].astype(jnp.int32)
    dst = edge_index[1].astype(jnp.int32)
    w = edge_weight[:, 0].astype(jnp.float32)

    eidx = jnp.arange(E_K, dtype=jnp.int32)
    dst_s, perm = lax.sort((dst, eidx), num_keys=1)
    src_s = jnp.take(src, perm)
    w_s = jnp.take(w, perm)
    bounds = jnp.arange(33, dtype=jnp.int32) * NPT
    starts = jnp.searchsorted(dst_s, bounds).astype(jnp.int32)
    starts48 = jnp.concatenate([starts, jnp.full((15,), E_K, jnp.int32)])

    edata = jnp.concatenate(
        [dst_s.reshape(NCHUNK, CHUNK),
         src_s.reshape(NCHUNK, CHUNK)], axis=1)
    edata = jnp.concatenate(
        [edata, jnp.zeros((SBC, EROW), jnp.int32)], axis=0).reshape(-1)
    wdata = jnp.concatenate([w_s, jnp.zeros((SBC * CHUNK,), jnp.float32)])

    ego = _build_ego(interaction_preference, interaction_embedding)
    ego = _sc_layer(ego, edata, wdata, starts48)
    ego = _sc_layer(ego, edata, wdata, starts48)
    return ego[:N_NODES_K]
